# Initial kernel scaffold; baseline (speedup 1.0000x reference)
import jax
import jax.numpy as jnp
from jax.experimental import pallas as pl


def kernel(x, edge_index, edge_attr, params):
    raise NotImplementedError("write your pallas kernel here")


# speedup vs baseline: 2.9655x; 2.9655x over previous
import functools

import jax
import jax.numpy as jnp
from jax import lax
from jax.experimental import pallas as pl
from jax.experimental.pallas import tpu as pltpu
from jax.experimental.pallas import tpu_sc as plsc

N = 10000
E = 320000
D = 128

NW = 32
EPW = E // NW
K = 80
NCHUNK = EPW // K
NPAD = 10240
RPT = NPAD // 16

_mesh = plsc.VectorSubcoreMesh(core_axis_name="c", subcore_axis_name="s")


@functools.partial(
    pl.kernel,
    out_type=(
        jax.ShapeDtypeStruct((E, D), jnp.float32),
        jax.ShapeDtypeStruct((E, D), jnp.float32),
    ),
    mesh=_mesh,
    scratch_types=[
        pltpu.VMEM((K,), jnp.int32),
        pltpu.VMEM((K,), jnp.int32),
        pltpu.VMEM((K, D), jnp.float32),
        pltpu.VMEM((K, D), jnp.float32),
        pltpu.SemaphoreType.DMA,
        pltpu.SemaphoreType.DMA,
    ],
)
def _sc_gather(p_hbm, q_hbm, didx_hbm, sidx_hbm, pd_hbm, qs_hbm,
               di_v, si_v, pr_v, qr_v, sem_p, sem_q):
    c = lax.axis_index("c")
    s = lax.axis_index("s")
    w = s * 2 + c

    def body(j, carry):
        base = (w * NCHUNK + j) * K
        pltpu.sync_copy(didx_hbm.at[w, j], di_v)
        pltpu.sync_copy(sidx_hbm.at[w, j], si_v)
        cp_p = pltpu.async_copy(p_hbm.at[di_v], pr_v, sem_p)
        cp_q = pltpu.async_copy(q_hbm.at[si_v], qr_v, sem_q)
        cp_p.wait()
        cp_q.wait()
        pltpu.sync_copy(pr_v, pd_hbm.at[pl.ds(base, K)])
        pltpu.sync_copy(qr_v, qs_hbm.at[pl.ds(base, K)])
        return carry

    lax.fori_loop(0, NCHUNK, body, 0)


@functools.partial(
    pl.kernel,
    out_type=jax.ShapeDtypeStruct((2, NPAD, D), jnp.float32),
    mesh=_mesh,
    scratch_types=[
        pltpu.VMEM((K,), jnp.int32),
        pltpu.VMEM((K, D), jnp.float32),
        pltpu.VMEM_SHARED((NPAD, D), jnp.float32),
    ],
)
def _sc_scatter(ue_hbm, sidx_hbm, zero_hbm, out_hbm, si_v, rows_v, acc):
    c = lax.axis_index("c")
    s = lax.axis_index("s")
    w = s * 2 + c
    r0 = s * RPT
    pltpu.sync_copy(zero_hbm.at[pl.ds(r0, RPT)], acc.at[pl.ds(r0, RPT)])
    plsc.subcore_barrier()

    def body(j, carry):
        base = (w * NCHUNK + j) * K
        pltpu.sync_copy(sidx_hbm.at[w, j], si_v)
        pltpu.sync_copy(ue_hbm.at[pl.ds(base, K)], rows_v)
        pltpu.sync_copy(rows_v, acc.at[si_v], add=True)
        return carry

    lax.fori_loop(0, NCHUNK, body, 0)
    plsc.subcore_barrier()
    pltpu.sync_copy(acc.at[pl.ds(r0, RPT)], out_hbm.at[c, pl.ds(r0, RPT)])


def _ln(h, g_ref, b_ref):
    m = jnp.mean(h, axis=-1, keepdims=True)
    d = h - m
    var = jnp.mean(d * d, axis=-1, keepdims=True)
    return d * lax.rsqrt(var + 1e-5) * g_ref[...] + b_ref[...]


def _edge_body(pd_ref, qs_ref, ea_ref, wet_ref, eb0_ref, ew1t_ref, eb1_ref,
               eg_ref, ebe_ref, ue_ref):
    ea = ea_ref[...]
    h = (pd_ref[...] + qs_ref[...]
         + jnp.dot(ea, wet_ref[...], preferred_element_type=jnp.float32)
         + eb0_ref[...])
    h = jnp.maximum(h, 0.0)
    h = jnp.dot(h, ew1t_ref[...], preferred_element_type=jnp.float32) + eb1_ref[...]
    ue_ref[...] = ea + _ln(h, eg_ref, ebe_ref)


EB = 2560

_full = lambda i: (0, 0)
_blk = lambda i: (i, 0)

_edge_call = pl.pallas_call(
    _edge_body,
    grid=(E // EB,),
    in_specs=[
        pl.BlockSpec((EB, D), _blk),
        pl.BlockSpec((EB, D), _blk),
        pl.BlockSpec((EB, D), _blk),
        pl.BlockSpec((D, D), _full),
        pl.BlockSpec((1, D), _full),
        pl.BlockSpec((D, D), _full),
        pl.BlockSpec((1, D), _full),
        pl.BlockSpec((1, D), _full),
        pl.BlockSpec((1, D), _full),
    ],
    out_specs=pl.BlockSpec((EB, D), _blk),
    out_shape=jax.ShapeDtypeStruct((E, D), jnp.float32),
)


def _node_body(x_ref, o0_ref, o1_ref, at_ref, bt_ref, nb0_ref, n1t_ref,
               nb1_ref, ng_ref, nbe_ref, wit_ref, wjt_ref,
               xo_ref, p_ref, q_ref):
    x = x_ref[...]
    o = o0_ref[...] + o1_ref[...]
    g = (jnp.dot(x, at_ref[...], preferred_element_type=jnp.float32)
         + jnp.dot(o, bt_ref[...], preferred_element_type=jnp.float32)
         + nb0_ref[...])
    g = jnp.maximum(g, 0.0)
    g = jnp.dot(g, n1t_ref[...], preferred_element_type=jnp.float32) + nb1_ref[...]
    xn = x + _ln(g, ng_ref, nbe_ref)
    xo_ref[...] = xn
    p_ref[...] = jnp.dot(xn, wit_ref[...], preferred_element_type=jnp.float32)
    q_ref[...] = jnp.dot(xn, wjt_ref[...], preferred_element_type=jnp.float32)


NB = 2000

_node_call = pl.pallas_call(
    _node_body,
    grid=(N // NB,),
    in_specs=[
        pl.BlockSpec((NB, D), _blk),
        pl.BlockSpec((NB, D), _blk),
        pl.BlockSpec((NB, D), _blk),
        pl.BlockSpec((D, D), _full),
        pl.BlockSpec((D, D), _full),
        pl.BlockSpec((1, D), _full),
        pl.BlockSpec((D, D), _full),
        pl.BlockSpec((1, D), _full),
        pl.BlockSpec((1, D), _full),
        pl.BlockSpec((1, D), _full),
        pl.BlockSpec((D, D), _full),
        pl.BlockSpec((D, D), _full),
    ],
    out_specs=[
        pl.BlockSpec((NB, D), _blk),
        pl.BlockSpec((NB, D), _blk),
        pl.BlockSpec((NB, D), _blk),
    ],
    out_shape=[
        jax.ShapeDtypeStruct((N, D), jnp.float32),
        jax.ShapeDtypeStruct((N, D), jnp.float32),
        jax.ShapeDtypeStruct((N, D), jnp.float32),
    ],
)


def _pq_body(x_ref, wit_ref, wjt_ref, p_ref, q_ref):
    x = x_ref[...]
    p_ref[...] = jnp.dot(x, wit_ref[...], preferred_element_type=jnp.float32)
    q_ref[...] = jnp.dot(x, wjt_ref[...], preferred_element_type=jnp.float32)


_pq_call = pl.pallas_call(
    _pq_body,
    grid=(N // NB,),
    in_specs=[
        pl.BlockSpec((NB, D), _blk),
        pl.BlockSpec((D, D), _full),
        pl.BlockSpec((D, D), _full),
    ],
    out_specs=[
        pl.BlockSpec((NB, D), _blk),
        pl.BlockSpec((NB, D), _blk),
    ],
    out_shape=[
        jax.ShapeDtypeStruct((N, D), jnp.float32),
        jax.ShapeDtypeStruct((N, D), jnp.float32),
    ],
)


def kernel(x, edge_index, edge_attr, params):
    src = edge_index[0].astype(jnp.int32)
    dst = edge_index[1].astype(jnp.int32)
    src3 = src.reshape(NW, NCHUNK, K)
    dst3 = dst.reshape(NW, NCHUNK, K)
    zeros = jnp.zeros((NPAD, D), jnp.float32)

    p0 = params[0]
    P, Q = _pq_call(x, p0['ew0'][:, :D].T, p0['ew0'][:, D:2 * D].T)

    ea = edge_attr
    for li, p in enumerate(params):
        Pd, Qs = _sc_gather(P, Q, dst3, src3)
        ue = _edge_call(Pd, Qs, ea,
                        p['ew0'][:, 2 * D:].T, p['eb0'][None],
                        p['ew1'].T, p['eb1'][None],
                        p['eg'][None], p['ebeta'][None])
        parts = _sc_scatter(ue, src3, zeros)
        parts = parts[:, :N]
        if li + 1 < len(params):
            nxt = params[li + 1]
            wit, wjt = nxt['ew0'][:, :D].T, nxt['ew0'][:, D:2 * D].T
        else:
            wit = wjt = jnp.zeros((D, D), jnp.float32)
        x, P, Q = _node_call(x, parts[0], parts[1],
                             p['nw0'][:, :D].T, p['nw0'][:, D:].T,
                             p['nb0'][None], p['nw1'].T, p['nb1'][None],
                             p['ng'][None], p['nbeta'][None], wit, wjt)
        ea = ue
    return x, ea


# speedup vs baseline: 4.5510x; 1.5346x over previous
import functools

import jax
import jax.numpy as jnp
from jax import lax
from jax.experimental import pallas as pl
from jax.experimental.pallas import tpu as pltpu
from jax.experimental.pallas import tpu_sc as plsc

N = 10000
E = 320000
D = 128

NW = 32
EPW = E // NW
K = 80
NCHUNK = EPW // K
NPAD = 10240
RPT = NPAD // 16

_mesh = plsc.VectorSubcoreMesh(core_axis_name="c", subcore_axis_name="s")


@functools.partial(
    pl.kernel,
    out_type=(
        jax.ShapeDtypeStruct((E, D), jnp.float32),
        jax.ShapeDtypeStruct((E, D), jnp.float32),
    ),
    mesh=_mesh,
    scratch_types=[
        pltpu.VMEM((NCHUNK, K), jnp.int32),
        pltpu.VMEM((NCHUNK, K), jnp.int32),
        pltpu.VMEM((K, D), jnp.float32),
        pltpu.VMEM((K, D), jnp.float32),
        pltpu.VMEM((K, D), jnp.float32),
        pltpu.VMEM((K, D), jnp.float32),
        pltpu.SemaphoreType.DMA,
        pltpu.SemaphoreType.DMA,
    ],
)
def _sc_gather(p_hbm, q_hbm, didx_hbm, sidx_hbm, pd_hbm, qs_hbm,
               di_all, si_all, pr_a, qr_a, pr_b, qr_b, sem_a, sem_b):
    c = lax.axis_index("c")
    s = lax.axis_index("s")
    w = s * 2 + c

    pltpu.sync_copy(didx_hbm.at[w], di_all)
    pltpu.sync_copy(sidx_hbm.at[w], si_all)

    def start(j, pr, qr, sem):
        pltpu.async_copy(p_hbm.at[di_all.at[j]], pr, sem)
        pltpu.async_copy(q_hbm.at[si_all.at[j]], qr, sem)

    def drain(pr, qr, sem):
        pltpu.make_async_copy(pd_hbm.at[pl.ds(0, K)], pr, sem).wait()
        pltpu.make_async_copy(qs_hbm.at[pl.ds(0, K)], qr, sem).wait()

    def out(j, pr, qr):
        base = (w * NCHUNK + j) * K
        pltpu.sync_copy(pr, pd_hbm.at[pl.ds(base, K)])
        pltpu.sync_copy(qr, qs_hbm.at[pl.ds(base, K)])

    start(0, pr_a, qr_a, sem_a)

    def body(t, carry):
        j0 = 2 * t
        start(j0 + 1, pr_b, qr_b, sem_b)
        drain(pr_a, qr_a, sem_a)
        out(j0, pr_a, qr_a)
        start(j0 + 2, pr_a, qr_a, sem_a)
        drain(pr_b, qr_b, sem_b)
        out(j0 + 1, pr_b, qr_b)
        return carry

    lax.fori_loop(0, NCHUNK // 2, body, 0)
    last = NCHUNK - 1
    drain(pr_a, qr_a, sem_a)
    out(last, pr_a, qr_a)


@functools.partial(
    pl.kernel,
    out_type=jax.ShapeDtypeStruct((2, NPAD, D), jnp.float32),
    mesh=_mesh,
    scratch_types=[
        pltpu.VMEM((NCHUNK, K), jnp.int32),
        pltpu.VMEM((K, D), jnp.float32),
        pltpu.VMEM((K, D), jnp.float32),
        pltpu.VMEM_SHARED((NPAD, D), jnp.float32),
        pltpu.SemaphoreType.DMA,
        pltpu.SemaphoreType.DMA,
    ],
)
def _sc_scatter(ue_hbm, sidx_hbm, zero_hbm, out_hbm, si_all, row_a, row_b,
                acc, sem_a, sem_b):
    c = lax.axis_index("c")
    s = lax.axis_index("s")
    w = s * 2 + c
    r0 = s * RPT
    pltpu.sync_copy(zero_hbm.at[pl.ds(r0, RPT)], acc.at[pl.ds(r0, RPT)])
    pltpu.sync_copy(sidx_hbm.at[w], si_all)
    plsc.subcore_barrier()

    def fetch(j, row, sem):
        base = (w * NCHUNK + j) * K
        pltpu.async_copy(ue_hbm.at[pl.ds(base, K)], row, sem)

    def scat(j, row, sem):
        pltpu.make_async_copy(ue_hbm.at[pl.ds(0, K)], row, sem).wait()
        pltpu.sync_copy(row, acc.at[si_all.at[j]], add=True)

    fetch(0, row_a, sem_a)

    def body(t, carry):
        j0 = 2 * t
        fetch(j0 + 1, row_b, sem_b)
        scat(j0, row_a, sem_a)
        fetch(j0 + 2, row_a, sem_a)
        scat(j0 + 1, row_b, sem_b)
        return carry

    lax.fori_loop(0, NCHUNK // 2, body, 0)
    scat(NCHUNK - 1, row_a, sem_a)
    plsc.subcore_barrier()
    pltpu.sync_copy(acc.at[pl.ds(r0, RPT)], out_hbm.at[c, pl.ds(r0, RPT)])


def _ln(h, g_ref, b_ref):
    m = jnp.mean(h, axis=-1, keepdims=True)
    d = h - m
    var = jnp.mean(d * d, axis=-1, keepdims=True)
    return d * lax.rsqrt(var + 1e-5) * g_ref[...] + b_ref[...]


def _edge_body(pd_ref, qs_ref, ea_ref, wet_ref, eb0_ref, ew1t_ref, eb1_ref,
               eg_ref, ebe_ref, ue_ref):
    ea = ea_ref[...]
    h = (pd_ref[...] + qs_ref[...]
         + jnp.dot(ea, wet_ref[...], preferred_element_type=jnp.float32)
         + eb0_ref[...])
    h = jnp.maximum(h, 0.0)
    h = jnp.dot(h, ew1t_ref[...], preferred_element_type=jnp.float32) + eb1_ref[...]
    ue_ref[...] = ea + _ln(h, eg_ref, ebe_ref)


EB = 2560

_full = lambda i: (0, 0)
_blk = lambda i: (i, 0)

_edge_call = pl.pallas_call(
    _edge_body,
    grid=(E // EB,),
    in_specs=[
        pl.BlockSpec((EB, D), _blk),
        pl.BlockSpec((EB, D), _blk),
        pl.BlockSpec((EB, D), _blk),
        pl.BlockSpec((D, D), _full),
        pl.BlockSpec((1, D), _full),
        pl.BlockSpec((D, D), _full),
        pl.BlockSpec((1, D), _full),
        pl.BlockSpec((1, D), _full),
        pl.BlockSpec((1, D), _full),
    ],
    out_specs=pl.BlockSpec((EB, D), _blk),
    out_shape=jax.ShapeDtypeStruct((E, D), jnp.float32),
)


def _node_body(x_ref, o0_ref, o1_ref, at_ref, bt_ref, nb0_ref, n1t_ref,
               nb1_ref, ng_ref, nbe_ref, wit_ref, wjt_ref,
               xo_ref, p_ref, q_ref):
    x = x_ref[...]
    o = o0_ref[...] + o1_ref[...]
    g = (jnp.dot(x, at_ref[...], preferred_element_type=jnp.float32)
         + jnp.dot(o, bt_ref[...], preferred_element_type=jnp.float32)
         + nb0_ref[...])
    g = jnp.maximum(g, 0.0)
    g = jnp.dot(g, n1t_ref[...], preferred_element_type=jnp.float32) + nb1_ref[...]
    xn = x + _ln(g, ng_ref, nbe_ref)
    xo_ref[...] = xn
    p_ref[...] = jnp.dot(xn, wit_ref[...], preferred_element_type=jnp.float32)
    q_ref[...] = jnp.dot(xn, wjt_ref[...], preferred_element_type=jnp.float32)


NB = 2000

_node_call = pl.pallas_call(
    _node_body,
    grid=(N // NB,),
    in_specs=[
        pl.BlockSpec((NB, D), _blk),
        pl.BlockSpec((NB, D), _blk),
        pl.BlockSpec((NB, D), _blk),
        pl.BlockSpec((D, D), _full),
        pl.BlockSpec((D, D), _full),
        pl.BlockSpec((1, D), _full),
        pl.BlockSpec((D, D), _full),
        pl.BlockSpec((1, D), _full),
        pl.BlockSpec((1, D), _full),
        pl.BlockSpec((1, D), _full),
        pl.BlockSpec((D, D), _full),
        pl.BlockSpec((D, D), _full),
    ],
    out_specs=[
        pl.BlockSpec((NB, D), _blk),
        pl.BlockSpec((NB, D), _blk),
        pl.BlockSpec((NB, D), _blk),
    ],
    out_shape=[
        jax.ShapeDtypeStruct((N, D), jnp.float32),
        jax.ShapeDtypeStruct((N, D), jnp.float32),
        jax.ShapeDtypeStruct((N, D), jnp.float32),
    ],
)


def _pq_body(x_ref, wit_ref, wjt_ref, p_ref, q_ref):
    x = x_ref[...]
    p_ref[...] = jnp.dot(x, wit_ref[...], preferred_element_type=jnp.float32)
    q_ref[...] = jnp.dot(x, wjt_ref[...], preferred_element_type=jnp.float32)


_pq_call = pl.pallas_call(
    _pq_body,
    grid=(N // NB,),
    in_specs=[
        pl.BlockSpec((NB, D), _blk),
        pl.BlockSpec((D, D), _full),
        pl.BlockSpec((D, D), _full),
    ],
    out_specs=[
        pl.BlockSpec((NB, D), _blk),
        pl.BlockSpec((NB, D), _blk),
    ],
    out_shape=[
        jax.ShapeDtypeStruct((N, D), jnp.float32),
        jax.ShapeDtypeStruct((N, D), jnp.float32),
    ],
)


def kernel(x, edge_index, edge_attr, params):
    src = edge_index[0].astype(jnp.int32)
    dst = edge_index[1].astype(jnp.int32)
    src3 = src.reshape(NW, NCHUNK, K)
    dst3 = dst.reshape(NW, NCHUNK, K)
    zeros = jnp.zeros((NPAD, D), jnp.float32)

    p0 = params[0]
    P, Q = _pq_call(x, p0['ew0'][:, :D].T, p0['ew0'][:, D:2 * D].T)

    ea = edge_attr
    for li, p in enumerate(params):
        Pd, Qs = _sc_gather(P, Q, dst3, src3)
        ue = _edge_call(Pd, Qs, ea,
                        p['ew0'][:, 2 * D:].T, p['eb0'][None],
                        p['ew1'].T, p['eb1'][None],
                        p['eg'][None], p['ebeta'][None])
        parts = _sc_scatter(ue, src3, zeros)
        parts = parts[:, :N]
        if li + 1 < len(params):
            nxt = params[li + 1]
            wit, wjt = nxt['ew0'][:, :D].T, nxt['ew0'][:, D:2 * D].T
        else:
            wit = wjt = jnp.zeros((D, D), jnp.float32)
        x, P, Q = _node_call(x, parts[0], parts[1],
                             p['nw0'][:, :D].T, p['nw0'][:, D:].T,
                             p['nb0'][None], p['nw1'].T, p['nb1'][None],
                             p['ng'][None], p['nbeta'][None], wit, wjt)
        ea = ue
    return x, ea
